# baseline (speedup 1.0000x reference)
import functools

import jax
import jax.numpy as jnp
from jax import lax
from jax.experimental import pallas as pl
from jax.experimental.pallas import tpu as pltpu
from jax.experimental.pallas import tpu_sc as plsc

N = 10000
E = 160000
IN_DIM = 128
HID = 64
HEADS = 6
HC = 384
OUT_DIM = 32
NG = 64
HALF = 192

NS = 16
EPAD = 163840
SCH = 2048
GCH = 128
BN = 1000
BE = 2048
N8 = N * 8

_f32 = jnp.float32
_bf16 = jnp.bfloat16
_i32 = jnp.int32


def _mesh():
    return plsc.VectorSubcoreMesh(core_axis_name="c", subcore_axis_name="s")


_SC_PARAMS = pltpu.CompilerParams(needs_layout_passes=False,
                                  use_tc_tiling_on_sc=False)


_RCH = 80
_NRC = N // _RCH


def _seg8_build(with_init, gather_back, interpret=False):
    outs = [jax.ShapeDtypeStruct((N, 8), _f32)]
    if gather_back:
        outs.append(jax.ShapeDtypeStruct((EPAD * 8,), _f32))

    def body(*refs):
        if with_init and gather_back:
            (vals_h, idx_h, zeros_h, ridx_h, init_h, den_h, denE_h,
             table, vbuf, ibuf, rbuf, rsem, den_sh) = refs
        elif not with_init and not gather_back:
            (vals_h, idx_h, zeros_h, ridx_h, den_h,
             table, vbuf, ibuf, rbuf, rsem, den_sh) = refs
            init_h = denE_h = None
        else:
            raise NotImplementedError
        cid = lax.axis_index("c")
        sid = lax.axis_index("s")
        pat01 = jnp.where(lax.iota(_i32, 16) < 8, 0, 1)
        mod8 = lax.rem(lax.iota(_i32, 16), 8)
        iota16 = lax.iota(_i32, 16)

        if True:
            @pl.when(cid == 0)
            def _():
                pltpu.sync_copy(zeros_h, table)

                @pl.when(sid == 0)
                def _():
                    pltpu.sync_copy(init_h if with_init else zeros_h, den_sh)

                nper = EPAD // NS
                base = sid * nper

                def chunk(ci, _):
                    cb = base + ci * SCH
                    pltpu.sync_copy(idx_h.at[pl.ds(cb, SCH)], ibuf)
                    pltpu.sync_copy(vals_h.at[pl.ds(cb * 8, SCH * 8)], vbuf)

                    def grp(gg, _):
                        for u in range(8):
                            g = gg * 8 + u
                            dv8 = plsc.load_gather(ibuf, [g * 2 + pat01])
                            plsc.addupdate_scatter(
                                table, [dv8, mod8], vbuf[pl.ds(g * 16, 16)])
                        return 0
                    lax.fori_loop(0, SCH // 16, grp, 0)
                    return 0
                lax.fori_loop(0, nper // SCH, chunk, 0)

            plsc.subcore_barrier()

            @pl.when(cid == 0)
            def _():
                pltpu.sync_copy(ridx_h, rbuf)

                def red(rc, _):
                    pltpu.async_copy(table.at[pl.ds(rc * _RCH, _RCH)],
                                     den_sh.at[rbuf.at[rc]], rsem, add=True)
                    return 0
                lax.fori_loop(0, _NRC, red, 0)

                def red_drain(rc, _):
                    pltpu.make_async_copy(
                        table.at[pl.ds(0, _RCH)],
                        den_sh.at[rbuf.at[0]], rsem).wait()
                    return 0
                lax.fori_loop(0, _NRC, red_drain, 0)

            plsc.subcore_barrier()

            @pl.when((cid == 0) & (sid == 0))
            def _():
                pltpu.sync_copy(den_sh, den_h)

            if gather_back:
                @pl.when(cid == 0)
                def _():
                    pltpu.sync_copy(den_sh, table)
                    nper = EPAD // NS
                    base = sid * nper

                    def chunk(ci, _):
                        cb = base + ci * SCH
                        pltpu.sync_copy(idx_h.at[pl.ds(cb, SCH)], ibuf)

                        def grp(gg, _):
                            for u in range(8):
                                g = gg * 8 + u
                                dv8 = plsc.load_gather(ibuf, [g * 2 + pat01])
                                vbuf[pl.ds(g * 16, 16)] = plsc.load_gather(
                                    table, [dv8, mod8])
                            return 0
                        lax.fori_loop(0, SCH // 16, grp, 0)
                        pltpu.sync_copy(vbuf, denE_h.at[pl.ds(cb * 8, SCH * 8)])
                        return 0
                    lax.fori_loop(0, nper // SCH, chunk, 0)

    return pl.kernel(
        body,
        out_type=tuple(outs) if len(outs) > 1 else outs[0],
        mesh=_mesh(),
        scratch_types=[
            pltpu.VMEM((N, 8), _f32),
            pltpu.VMEM((SCH * 8,), _f32),
            pltpu.VMEM((SCH,), _i32),
            pltpu.VMEM((_NRC, _RCH), _i32),
            pltpu.SemaphoreType.DMA,
            pltpu.VMEM_SHARED((N, 8), _f32),
        ],
        compiler_params=_SC_PARAMS,
        interpret=interpret,
    )


def _gather2_build(interpret=False):
    outs = tuple(jax.ShapeDtypeStruct((EPAD, HALF), _bf16) for _ in range(4))

    def body(xlA, xlB, xrA, xrB, src_h, dst_h, glA, glB, grA, grB,
             ibig_s, ibig_d, rb0, rb1, rb2, rb3,
             gs0, gs1, gs2, gs3, ws0, ws1, ws2, ws3):
        cid = lax.axis_index("c")
        sid = lax.axis_index("s")
        nper = EPAD // NS
        base = sid * nper
        nch = nper // GCH
        rbufs = (rb0, rb1, rb2, rb3)
        gsems = (gs0, gs1, gs2, gs3)
        wsems = (ws0, ws1, ws2, ws3)

        def do_half(tab_l, tab_r, out_l, out_r):
            pltpu.sync_copy(src_h.at[pl.ds(base, nper)], ibig_s)
            pltpu.sync_copy(dst_h.at[pl.ds(base, nper)], ibig_d)

            for tab, ibig, out in ((tab_l, ibig_s, out_l),
                                   (tab_r, ibig_d, out_r)):
                def gather_start(ci, b):
                    pltpu.async_copy(tab.at[ibig.at[pl.ds(ci * GCH, GCH)]],
                                     rbufs[b], gsems[b])

                def gather_wait(b):
                    pltpu.make_async_copy(tab.at[pl.ds(0, GCH)], rbufs[b],
                                          gsems[b]).wait()

                def write_start(ci, b):
                    pltpu.async_copy(rbufs[b],
                                     out.at[pl.ds(base + ci * GCH, GCH)],
                                     wsems[b])

                def write_wait(b):
                    pltpu.make_async_copy(tab.at[pl.ds(0, GCH)], rbufs[b],
                                          wsems[b]).wait()

                gather_start(0, 0)

                def outer(i4, _):
                    for b in range(4):
                        ci = i4 * 4 + b
                        bn = (b + 1) % 4

                        @pl.when(ci >= 3)
                        def _():
                            write_wait(bn)

                        @pl.when(ci + 1 < nch)
                        def _():
                            gather_start(ci + 1, bn)
                        gather_wait(b)
                        write_start(ci, b)
                    return 0
                lax.fori_loop(0, nch // 4, outer, 0)
                write_wait(1)
                write_wait(2)
                write_wait(3)

        @pl.when(cid == 0)
        def _():
            do_half(xlA, xrA, glA, grA)

        @pl.when(cid == 1)
        def _():
            do_half(xlB, xrB, glB, grB)

    return pl.kernel(
        body,
        out_type=outs,
        mesh=_mesh(),
        scratch_types=[
            pltpu.VMEM((EPAD // NS,), _i32),
            pltpu.VMEM((EPAD // NS,), _i32),
            pltpu.VMEM((GCH, HALF), _bf16),
            pltpu.VMEM((GCH, HALF), _bf16),
            pltpu.VMEM((GCH, HALF), _bf16),
            pltpu.VMEM((GCH, HALF), _bf16),
            pltpu.SemaphoreType.DMA,
            pltpu.SemaphoreType.DMA,
            pltpu.SemaphoreType.DMA,
            pltpu.SemaphoreType.DMA,
            pltpu.SemaphoreType.DMA,
            pltpu.SemaphoreType.DMA,
            pltpu.SemaphoreType.DMA,
            pltpu.SemaphoreType.DMA,
        ],
        compiler_params=_SC_PARAMS,
        interpret=interpret,
    )


QC = 96


def _scatter_rows_build(interpret=False):
    outs = tuple(jax.ShapeDtypeStruct((N, QC), _f32) for _ in range(2))

    def body(wA, wB, dst2_h, initA, initB, accA, accB,
             ibig, rb0, rb1, rb2, rb3, ls0, ls1, ls2, ls3,
             ss0, ss1, ss2, ss3, acc_sh):
        cid = lax.axis_index("c")
        sid = lax.axis_index("s")
        rows_per = N // NS
        nch = (EPAD // NS) // GCH
        cbase = sid * nch
        rbufs = (rb0, rb1, rb2, rb3)
        lsems = (ls0, ls1, ls2, ls3)
        ssems = (ss0, ss1, ss2, ss3)

        def do_half(w_h, init_h, out_h):
            pltpu.sync_copy(init_h.at[pl.ds(sid * rows_per, rows_per)],
                            acc_sh.at[pl.ds(sid * rows_per, rows_per)])
            pltpu.sync_copy(dst2_h.at[pl.ds(cbase, nch)], ibig)
            plsc.subcore_barrier()

            def load_start(ci, b):
                pltpu.async_copy(w_h.at[pl.ds((cbase + ci) * GCH, GCH)],
                                 rbufs[b], lsems[b])

            def load_wait(b):
                pltpu.make_async_copy(w_h.at[pl.ds(0, GCH)], rbufs[b],
                                      lsems[b]).wait()

            def scat_start(ci, b):
                pltpu.async_copy(rbufs[b], acc_sh.at[ibig.at[ci]], ssems[b],
                                 add=True)

            def scat_wait(b):
                pltpu.make_async_copy(w_h.at[pl.ds(0, GCH)], rbufs[b],
                                      ssems[b]).wait()

            load_start(0, 0)

            def outer(i4, _):
                for b in range(4):
                    ci = i4 * 4 + b
                    bn = (b + 1) % 4

                    @pl.when(ci >= 3)
                    def _():
                        scat_wait(bn)

                    @pl.when(ci + 1 < nch)
                    def _():
                        load_start(ci + 1, bn)
                    load_wait(b)
                    scat_start(ci, b)
                return 0
            lax.fori_loop(0, nch // 4, outer, 0)
            scat_wait(1)
            scat_wait(2)
            scat_wait(3)
            plsc.subcore_barrier()
            pltpu.sync_copy(acc_sh.at[pl.ds(sid * rows_per, rows_per)],
                            out_h.at[pl.ds(sid * rows_per, rows_per)])

        @pl.when(cid == 0)
        def _():
            do_half(wA, initA, accA)

        @pl.when(cid == 1)
        def _():
            do_half(wB, initB, accB)

    return pl.kernel(
        body,
        out_type=outs,
        mesh=_mesh(),
        scratch_types=[
            pltpu.VMEM((EPAD // GCH // NS, GCH), _i32),
            pltpu.VMEM((GCH, QC), _f32),
            pltpu.VMEM((GCH, QC), _f32),
            pltpu.VMEM((GCH, QC), _f32),
            pltpu.VMEM((GCH, QC), _f32),
            pltpu.SemaphoreType.DMA,
            pltpu.SemaphoreType.DMA,
            pltpu.SemaphoreType.DMA,
            pltpu.SemaphoreType.DMA,
            pltpu.SemaphoreType.DMA,
            pltpu.SemaphoreType.DMA,
            pltpu.SemaphoreType.DMA,
            pltpu.SemaphoreType.DMA,
            pltpu.VMEM_SHARED((N, QC), _f32),
        ],
        compiler_params=_SC_PARAMS,
        interpret=interpret,
    )


def _head_logits(z, att, h0):
    cols = []
    for k in range(3):
        h = h0 + k
        zr = z[:, k * HID:(k + 1) * HID]
        cols.append(jnp.sum(zr * att[h:h + 1, :], axis=1, keepdims=True))
    return jnp.concatenate(cols, axis=1)


def _lrelu(x):
    return jnp.where(x >= 0, x, 0.2 * x)


def _m0_kernel(x_ref, ntm_ref, ls_ref, w4_ref, wl_ref, bl_ref, wr_ref,
               br_ref, we_ref, att_ref,
               xlA_ref, xlB_ref, xrA_ref, xrB_ref, ps_ref, la_ref):
    xb = x_ref[...]
    proj = jnp.dot(xb, w4_ref[...], preferred_element_type=_f32)
    t = ntm_ref[...]
    h0 = jnp.zeros((BN, HID), _f32)
    for k in range(4):
        h0 = jnp.where(t == float(k), proj[:, k * HID:(k + 1) * HID], h0)
    h0 = jnp.maximum(h0, 0.0)
    xl = jnp.dot(h0, wl_ref[...], preferred_element_type=_f32) + bl_ref[...]
    xr = jnp.dot(h0, wr_ref[...], preferred_element_type=_f32) + br_ref[...]
    ls = ls_ref[...]
    cnt = jnp.maximum(ls[:, 4:5], 1.0)
    la = ls / cnt
    la = jnp.where(jnp.arange(8)[None, :] < 4, la, 0.0)
    eemb = jnp.dot(la, we_ref[...], preferred_element_type=_f32)
    z = _lrelu(xl + xr + eemb)
    att = att_ref[...]
    lg = jnp.concatenate(
        [_head_logits(z[:, :HALF], att, 0), _head_logits(z[:, HALF:], att, 3)],
        axis=1)
    ps = jnp.exp(lg)
    ps_ref[...] = jnp.concatenate([ps, jnp.zeros((BN, 2), _f32)], axis=1)
    la_ref[...] = la
    xlA_ref[...] = xl[:, :HALF].astype(_bf16)
    xlB_ref[...] = xl[:, HALF:].astype(_bf16)
    xrA_ref[...] = xr[:, :HALF].astype(_bf16)
    xrB_ref[...] = xr[:, HALF:].astype(_bf16)


def _m1_kernel(e_const, glA_ref, glB_ref, grA_ref, grB_ref, ea_ref, we_ref,
               att_ref, p_ref):
    i = pl.program_id(0)
    ea = ea_ref[...]
    att = att_ref[...]
    we = we_ref[...]
    zA = _lrelu(glA_ref[...].astype(_f32) + grA_ref[...].astype(_f32) +
                jnp.dot(ea, we[:, :HALF], preferred_element_type=_f32))
    zB = _lrelu(glB_ref[...].astype(_f32) + grB_ref[...].astype(_f32) +
                jnp.dot(ea, we[:, HALF:], preferred_element_type=_f32))
    lg = jnp.concatenate([_head_logits(zA, att, 0), _head_logits(zB, att, 3)],
                         axis=1)
    p = jnp.exp(lg)
    ids = i * BE + lax.broadcasted_iota(_i32, (BE, 1), 0)
    p = jnp.where(ids < e_const, p, 0.0)
    p_ref[...] = jnp.concatenate([p, jnp.zeros((BE, 2), _f32)], axis=1)


def _m2a_kernel(ps_ref, den_ref, xlA_ref, xlB_ref,
                iA0_ref, iA1_ref, iB0_ref, iB1_ref):
    a = ps_ref[...] / jnp.maximum(den_ref[...], 1e-30)
    xlA = xlA_ref[...].astype(_f32)
    xlB = xlB_ref[...].astype(_f32)
    iA = jnp.concatenate(
        [a[:, h:h + 1] * xlA[:, h * HID:(h + 1) * HID] for h in range(3)], axis=1)
    iB = jnp.concatenate(
        [a[:, 3 + h:4 + h] * xlB[:, h * HID:(h + 1) * HID] for h in range(3)],
        axis=1)
    iA0_ref[...] = iA[:, :QC]
    iA1_ref[...] = iA[:, QC:]
    iB0_ref[...] = iB[:, :QC]
    iB1_ref[...] = iB[:, QC:]


def _m2c_kernel(p_ref, denE_ref, glA_ref, glB_ref,
                wA0_ref, wA1_ref, wB0_ref, wB1_ref):
    a = p_ref[...] / jnp.maximum(denE_ref[...], 1e-30)
    glA = glA_ref[...].astype(_f32)
    glB = glB_ref[...].astype(_f32)
    wA = jnp.concatenate(
        [a[:, h:h + 1] * glA[:, h * HID:(h + 1) * HID] for h in range(3)], axis=1)
    wB = jnp.concatenate(
        [a[:, 3 + h:4 + h] * glB[:, h * HID:(h + 1) * HID] for h in range(3)],
        axis=1)
    wA0_ref[...] = wA[:, :QC]
    wA1_ref[...] = wA[:, QC:]
    wB0_ref[...] = wB[:, :QC]
    wB1_ref[...] = wB[:, QC:]


def _ml2_body(accA0_ref, accA1_ref, accB0_ref, accB1_ref, ls_ref, bias_ref,
              wl_ref, bl_ref, wr_ref, br_ref, we_ref, att_ref,
              xlA_ref, xlB_ref, xrA_ref, xrB_ref, ps_ref):
    bias = bias_ref[...]
    accA = jnp.concatenate([accA0_ref[...], accA1_ref[...]], axis=1)
    accB = jnp.concatenate([accB0_ref[...], accB1_ref[...]], axis=1)
    h1A = jnp.maximum(accA + bias[:, :HALF], 0.0)
    h1B = jnp.maximum(accB + bias[:, HALF:], 0.0)
    wl = wl_ref[...]
    wr = wr_ref[...]
    xl = (jnp.dot(h1A, wl[:HALF, :], preferred_element_type=_f32) +
          jnp.dot(h1B, wl[HALF:, :], preferred_element_type=_f32) + bl_ref[...])
    xr = (jnp.dot(h1A, wr[:HALF, :], preferred_element_type=_f32) +
          jnp.dot(h1B, wr[HALF:, :], preferred_element_type=_f32) + br_ref[...])
    la = ls_ref[...]
    eemb = jnp.dot(la, we_ref[...], preferred_element_type=_f32)
    z = _lrelu(xl + xr + eemb)
    att = att_ref[...]
    lg = jnp.concatenate(
        [_head_logits(z[:, :HALF], att, 0), _head_logits(z[:, HALF:], att, 3)],
        axis=1)
    ps_ref[...] = jnp.concatenate([jnp.exp(lg), jnp.zeros((BN, 2), _f32)],
                                  axis=1)
    xlA_ref[...] = xl[:, :HALF].astype(_bf16)
    xlB_ref[...] = xl[:, HALF:].astype(_bf16)
    xrA_ref[...] = xr[:, :HALF].astype(_bf16)
    xrB_ref[...] = xr[:, HALF:].astype(_bf16)


def _m3_kernel(accA0_ref, accA1_ref, accB0_ref, accB1_ref, bias_ref,
               batch_ref, wo_ref, bo_ref, out_ref, pA_ref, pB_ref, cnt_ref):
    i = pl.program_id(0)
    nb = pl.num_programs(0)

    @pl.when(i == 0)
    def _():
        pA_ref[...] = jnp.zeros_like(pA_ref)
        pB_ref[...] = jnp.zeros_like(pB_ref)
        cnt_ref[...] = jnp.zeros_like(cnt_ref)

    bias = bias_ref[...]
    accA = jnp.concatenate([accA0_ref[...], accA1_ref[...]], axis=1)
    accB = jnp.concatenate([accB0_ref[...], accB1_ref[...]], axis=1)
    h2A = jnp.maximum(accA + bias[:, :HALF], 0.0)
    h2B = jnp.maximum(accB + bias[:, HALF:], 0.0)
    b = batch_ref[...]
    gids = lax.broadcasted_iota(_i32, (1, NG), 1).astype(_f32)
    oh = (b == gids).astype(_f32)
    dn = (((0,), (0,)), ((), ()))
    pA_ref[...] += lax.dot_general(oh, h2A, dn, preferred_element_type=_f32)
    pB_ref[...] += lax.dot_general(oh, h2B, dn, preferred_element_type=_f32)
    cnt_ref[...] += jnp.sum(oh, axis=0)[:, None]

    @pl.when(i == nb - 1)
    def _():
        cnt = jnp.maximum(cnt_ref[...], 1.0)
        wo = wo_ref[...]
        out = (jnp.dot(pA_ref[...] / cnt, wo[:HALF, :],
                       preferred_element_type=_f32) +
               jnp.dot(pB_ref[...] / cnt, wo[HALF:, :],
                       preferred_element_type=_f32) + bo_ref[...])
        out_ref[...] = jnp.tanh(out)


_NB = N // BN
_EB = EPAD // BE


def _nspec(c):
    return pl.BlockSpec((BN, c), lambda i: (i, 0))


def _espec(c):
    return pl.BlockSpec((BE, c), lambda i: (i, 0))


def _full(shape):
    return pl.BlockSpec(shape, lambda i: tuple(0 for _ in shape))


def _m0_call(x, ntmf, ls, w4, wl, bl, wr, br, we, att, interpret=False):
    outs = (
        jax.ShapeDtypeStruct((N, HALF), _bf16),
        jax.ShapeDtypeStruct((N, HALF), _bf16),
        jax.ShapeDtypeStruct((N, HALF), _bf16),
        jax.ShapeDtypeStruct((N, HALF), _bf16),
        jax.ShapeDtypeStruct((N, 8), _f32),
        jax.ShapeDtypeStruct((N, 8), _f32),
    )
    return pl.pallas_call(
        _m0_kernel,
        grid=(_NB,),
        in_specs=[_nspec(IN_DIM), _nspec(1), _nspec(8), _full((IN_DIM, 4 * HID)),
                  _full((HID, HC)), _full((1, HC)), _full((HID, HC)),
                  _full((1, HC)), _full((8, HC)), _full((8, HID))],
        out_specs=(_nspec(HALF), _nspec(HALF), _nspec(HALF), _nspec(HALF),
                   _nspec(8), _nspec(8)),
        out_shape=outs,
        interpret=interpret,
    )(x, ntmf, ls, w4, wl, bl, wr, br, we, att)


def _m1_call(glA, glB, grA, grB, ea8, we, att, interpret=False):
    return pl.pallas_call(
        functools.partial(_m1_kernel, E),
        grid=(_EB,),
        in_specs=[_espec(HALF)] * 4 + [_espec(8), _full((8, HC)),
                                       _full((8, HID))],
        out_specs=_espec(8),
        out_shape=jax.ShapeDtypeStruct((EPAD, 8), _f32),
        interpret=interpret,
    )(glA, glB, grA, grB, ea8, we, att)


def _m2a_call(ps, den, xlA, xlB, interpret=False):
    outs = tuple(jax.ShapeDtypeStruct((N, QC), _f32) for _ in range(4))
    return pl.pallas_call(
        _m2a_kernel,
        grid=(_NB,),
        in_specs=[_nspec(8), _nspec(8), _nspec(HALF), _nspec(HALF)],
        out_specs=tuple(_nspec(QC) for _ in range(4)),
        out_shape=outs,
        interpret=interpret,
    )(ps, den, xlA, xlB)


def _m2c_call(p, denE, glA, glB, interpret=False):
    outs = tuple(jax.ShapeDtypeStruct((EPAD, QC), _f32) for _ in range(4))
    return pl.pallas_call(
        _m2c_kernel,
        grid=(_EB,),
        in_specs=[_espec(8), _espec(8), _espec(HALF), _espec(HALF)],
        out_specs=tuple(_espec(QC) for _ in range(4)),
        out_shape=outs,
        interpret=interpret,
    )(p, denE, glA, glB)


def _ml2_call(accs, ls, bias, wl, bl, wr, br, we, att, interpret=False):
    outs = tuple(jax.ShapeDtypeStruct((N, HALF), _bf16) for _ in range(4)) + (
        jax.ShapeDtypeStruct((N, 8), _f32),)
    return pl.pallas_call(
        _ml2_body,
        grid=(_NB,),
        in_specs=[_nspec(QC)] * 4 + [_nspec(8), _full((1, HC)),
                  _full((HC, HC)), _full((1, HC)), _full((HC, HC)),
                  _full((1, HC)), _full((8, HC)), _full((8, HID))],
        out_specs=(_nspec(HALF), _nspec(HALF), _nspec(HALF), _nspec(HALF),
                   _nspec(8)),
        out_shape=outs,
        interpret=interpret,
    )(*accs, ls, bias, wl, bl, wr, br, we, att)


def _m3_call(accs, bias, batchf, wo, bo, interpret=False):
    return pl.pallas_call(
        _m3_kernel,
        grid=(_NB,),
        in_specs=[_nspec(QC)] * 4 + [_full((1, HC)), _nspec(1),
                  _full((HC, OUT_DIM)), _full((1, OUT_DIM))],
        out_specs=_full((NG, OUT_DIM)),
        out_shape=jax.ShapeDtypeStruct((NG, OUT_DIM), _f32),
        scratch_shapes=[pltpu.VMEM((NG, HALF), _f32),
                        pltpu.VMEM((NG, HALF), _f32),
                        pltpu.VMEM((NG, 1), _f32)],
        interpret=interpret,
    )(*accs, bias, batchf, wo, bo)


def _layer(h_parts, src_p, dst_p, dst2, ridx, ea8, la, p, first, x=None,
           ntmf=None, ls=None, w4=None, interpret=False):
    wl, bl = p["Wl"], p["bl"][None, :]
    wr, br = p["Wr"], p["br"][None, :]
    we8 = jnp.pad(p["We"], ((0, 4), (0, 0)))
    att8 = jnp.pad(p["att"], ((0, 2), (0, 0)))
    if first:
        xlA, xlB, xrA, xrB, ps, la_out = _m0_call(
            x, ntmf, ls, w4, wl, bl, wr, br, we8, att8, interpret=interpret)
    else:
        accs, bias_prev = h_parts
        xlA, xlB, xrA, xrB, ps = _ml2_call(
            accs, la, bias_prev, wl, bl, wr, br, we8, att8,
            interpret=interpret)
        la_out = la
    glA, glB, grA, grB = _gather2_build(interpret=interpret)(
        xlA, xlB, xrA, xrB, src_p, dst_p)
    pmat = _m1_call(glA, glB, grA, grB, ea8, we8, att8, interpret=interpret)
    den, denE_flat = _seg8_build(True, True, interpret=interpret)(
        pmat.reshape(-1), dst_p, jnp.zeros((N, 8), _f32), ridx, ps)
    denE = denE_flat.reshape(EPAD, 8)
    iA0, iA1, iB0, iB1 = _m2a_call(ps, den, xlA, xlB, interpret=interpret)
    wA0, wA1, wB0, wB1 = _m2c_call(pmat, denE, glA, glB, interpret=interpret)
    scat = _scatter_rows_build(interpret=interpret)
    accA0, accB0 = scat(wA0, wB0, dst2, iA0, iB0)
    accA1, accB1 = scat(wA1, wB1, dst2, iA1, iB1)
    return (accA0, accA1, accB0, accB1), la_out


def kernel(x, edge_index, edge_attr, node_type_mask, batch, params):
    src = edge_index[0].astype(_i32)
    dst = edge_index[1].astype(_i32)
    src_p = jnp.pad(src, (0, EPAD - E))
    dst_p = jnp.pad(dst, (0, EPAD - E))
    ea8 = jnp.pad(edge_attr.astype(_f32), ((0, EPAD - E), (0, 4)))
    vals0 = jnp.pad(
        jnp.concatenate([edge_attr.astype(_f32),
                         jnp.ones((E, 1), _f32)], axis=1),
        ((0, EPAD - E), (0, 3)))
    ntmf = node_type_mask.astype(_f32)[:, None]
    batchf = batch.astype(_f32)[:, None]
    w4 = jnp.concatenate(
        [params["W_" + n] for n in ["joint", "obj", "tcp", "goal"]], axis=1)

    dst2 = dst_p.reshape(EPAD // GCH, GCH)
    ridx = jnp.arange(N, dtype=_i32).reshape(_NRC, _RCH)

    interpret = False
    ls = _seg8_build(False, False, interpret=interpret)(
        vals0.reshape(-1), dst_p, jnp.zeros((N, 8), _f32), ridx)

    c0, c1 = params["convs"]
    accs, la = _layer(None, src_p, dst_p, dst2, ridx, ea8, None, c0, True,
                      x=x, ntmf=ntmf, ls=ls, w4=w4, interpret=interpret)
    accs, _ = _layer((accs, c0["bias"][None, :]), src_p, dst_p, dst2, ridx,
                     ea8, la, c1, False, interpret=interpret)
    return _m3_call(accs, c1["bias"][None, :], batchf,
                    params["W_out"], params["b_out"][None, :],
                    interpret=interpret)


# speedup vs baseline: 1.1219x; 1.1219x over previous
import functools

import jax
import jax.numpy as jnp
from jax import lax
from jax.experimental import pallas as pl
from jax.experimental.pallas import tpu as pltpu
from jax.experimental.pallas import tpu_sc as plsc

N = 10000
E = 160000
IN_DIM = 128
HID = 64
HEADS = 6
HC = 384
OUT_DIM = 32
NG = 64
HALF = 192

NS = 16
EPAD = 163840
SCH = 2048
GCH = 128
BN = 1000
BE = 4096
N8 = N * 8

_f32 = jnp.float32
_bf16 = jnp.bfloat16
_i32 = jnp.int32


def _mesh():
    return plsc.VectorSubcoreMesh(core_axis_name="c", subcore_axis_name="s")


_SC_PARAMS = pltpu.CompilerParams(needs_layout_passes=False,
                                  use_tc_tiling_on_sc=False)


_RCH = 80
_NRC = N // _RCH


def _seg8_build(with_init, gather_back, interpret=False):
    outs = [jax.ShapeDtypeStruct((N, 8), _f32)]
    if gather_back:
        outs.append(jax.ShapeDtypeStruct((EPAD * 8,), _f32))

    def body(*refs):
        if with_init and gather_back:
            (vals_h, idx_h, zeros_h, ridx_h, init_h, den_h, denE_h,
             table, vbuf, ibuf, rbuf, rsem, den_sh) = refs
        elif not with_init and not gather_back:
            (vals_h, idx_h, zeros_h, ridx_h, den_h,
             table, vbuf, ibuf, rbuf, rsem, den_sh) = refs
            init_h = denE_h = None
        else:
            raise NotImplementedError
        cid = lax.axis_index("c")
        sid = lax.axis_index("s")
        pat01 = jnp.where(lax.iota(_i32, 16) < 8, 0, 1)
        mod8 = lax.rem(lax.iota(_i32, 16), 8)
        iota16 = lax.iota(_i32, 16)

        if True:
            @pl.when(cid == 0)
            def _():
                pltpu.sync_copy(zeros_h, table)

                @pl.when(sid == 0)
                def _():
                    pltpu.sync_copy(init_h if with_init else zeros_h, den_sh)

                nper = EPAD // NS
                base = sid * nper

                def chunk(ci, _):
                    cb = base + ci * SCH
                    pltpu.sync_copy(idx_h.at[pl.ds(cb, SCH)], ibuf)
                    pltpu.sync_copy(vals_h.at[pl.ds(cb * 8, SCH * 8)], vbuf)

                    def grp(gg, _):
                        for u in range(8):
                            g = gg * 8 + u
                            dv8 = plsc.load_gather(ibuf, [g * 2 + pat01])
                            plsc.addupdate_scatter(
                                table, [dv8, mod8], vbuf[pl.ds(g * 16, 16)])
                        return 0
                    lax.fori_loop(0, SCH // 16, grp, 0)
                    return 0
                lax.fori_loop(0, nper // SCH, chunk, 0)

            plsc.subcore_barrier()

            @pl.when(cid == 0)
            def _():
                pltpu.sync_copy(ridx_h, rbuf)

                def red(rc, _):
                    pltpu.async_copy(table.at[pl.ds(rc * _RCH, _RCH)],
                                     den_sh.at[rbuf.at[rc]], rsem, add=True)
                    return 0
                lax.fori_loop(0, _NRC, red, 0)

                def red_drain(rc, _):
                    pltpu.make_async_copy(
                        table.at[pl.ds(0, _RCH)],
                        den_sh.at[rbuf.at[0]], rsem).wait()
                    return 0
                lax.fori_loop(0, _NRC, red_drain, 0)

            plsc.subcore_barrier()

            @pl.when((cid == 0) & (sid == 0))
            def _():
                pltpu.sync_copy(den_sh, den_h)

            if gather_back:
                @pl.when(cid == 0)
                def _():
                    pltpu.sync_copy(den_sh, table)
                    nper = EPAD // NS
                    base = sid * nper

                    def chunk(ci, _):
                        cb = base + ci * SCH
                        pltpu.sync_copy(idx_h.at[pl.ds(cb, SCH)], ibuf)

                        def grp(gg, _):
                            for u in range(8):
                                g = gg * 8 + u
                                dv8 = plsc.load_gather(ibuf, [g * 2 + pat01])
                                vbuf[pl.ds(g * 16, 16)] = plsc.load_gather(
                                    table, [dv8, mod8])
                            return 0
                        lax.fori_loop(0, SCH // 16, grp, 0)
                        pltpu.sync_copy(vbuf, denE_h.at[pl.ds(cb * 8, SCH * 8)])
                        return 0
                    lax.fori_loop(0, nper // SCH, chunk, 0)

    return pl.kernel(
        body,
        out_type=tuple(outs) if len(outs) > 1 else outs[0],
        mesh=_mesh(),
        scratch_types=[
            pltpu.VMEM((N, 8), _f32),
            pltpu.VMEM((SCH * 8,), _f32),
            pltpu.VMEM((SCH,), _i32),
            pltpu.VMEM((_NRC, _RCH), _i32),
            pltpu.SemaphoreType.DMA,
            pltpu.VMEM_SHARED((N, 8), _f32),
        ],
        compiler_params=_SC_PARAMS,
        interpret=interpret,
    )


def _gather2_build(interpret=False):
    outs = tuple(jax.ShapeDtypeStruct((EPAD, HALF), _f32) for _ in range(4))

    def body(xlA, xlB, xrA, xrB, src_h, dst_h, glA, glB, grA, grB,
             ibig_s, ibig_d, rb0, rb1, rb2, rb3,
             gs0, gs1, gs2, gs3, ws0, ws1, ws2, ws3):
        cid = lax.axis_index("c")
        sid = lax.axis_index("s")
        nper = EPAD // NS
        base = sid * nper
        nch = nper // GCH
        rbufs = (rb0, rb1, rb2, rb3)
        gsems = (gs0, gs1, gs2, gs3)
        wsems = (ws0, ws1, ws2, ws3)

        def do_half(tab_l, tab_r, out_l, out_r):
            pltpu.sync_copy(src_h.at[pl.ds(base, nper)], ibig_s)
            pltpu.sync_copy(dst_h.at[pl.ds(base, nper)], ibig_d)

            for tab, ibig, out in ((tab_l, ibig_s, out_l),
                                   (tab_r, ibig_d, out_r)):
                def gather_start(ci, b):
                    pltpu.async_copy(tab.at[ibig.at[pl.ds(ci * GCH, GCH)]],
                                     rbufs[b], gsems[b])

                def gather_wait(b):
                    pltpu.make_async_copy(tab.at[pl.ds(0, GCH)], rbufs[b],
                                          gsems[b]).wait()

                def write_start(ci, b):
                    pltpu.async_copy(rbufs[b],
                                     out.at[pl.ds(base + ci * GCH, GCH)],
                                     wsems[b])

                def write_wait(b):
                    pltpu.make_async_copy(tab.at[pl.ds(0, GCH)], rbufs[b],
                                          wsems[b]).wait()

                gather_start(0, 0)

                def outer(i4, _):
                    for b in range(4):
                        ci = i4 * 4 + b
                        bn = (b + 1) % 4

                        @pl.when(ci >= 3)
                        def _():
                            write_wait(bn)

                        @pl.when(ci + 1 < nch)
                        def _():
                            gather_start(ci + 1, bn)
                        gather_wait(b)
                        write_start(ci, b)
                    return 0
                lax.fori_loop(0, nch // 4, outer, 0)
                write_wait(1)
                write_wait(2)
                write_wait(3)

        @pl.when(cid == 0)
        def _():
            do_half(xlA, xrA, glA, grA)

        @pl.when(cid == 1)
        def _():
            do_half(xlB, xrB, glB, grB)

    return pl.kernel(
        body,
        out_type=outs,
        mesh=_mesh(),
        scratch_types=[
            pltpu.VMEM((EPAD // NS,), _i32),
            pltpu.VMEM((EPAD // NS,), _i32),
            pltpu.VMEM((GCH, HALF), _f32),
            pltpu.VMEM((GCH, HALF), _f32),
            pltpu.VMEM((GCH, HALF), _f32),
            pltpu.VMEM((GCH, HALF), _f32),
            pltpu.SemaphoreType.DMA,
            pltpu.SemaphoreType.DMA,
            pltpu.SemaphoreType.DMA,
            pltpu.SemaphoreType.DMA,
            pltpu.SemaphoreType.DMA,
            pltpu.SemaphoreType.DMA,
            pltpu.SemaphoreType.DMA,
            pltpu.SemaphoreType.DMA,
        ],
        compiler_params=_SC_PARAMS,
        interpret=interpret,
    )


QC = 96


def _scatter_rows_build(interpret=False):
    outs = tuple(jax.ShapeDtypeStruct((N, QC), _f32) for _ in range(2))

    def body(wA, wB, dst2_h, initA, initB, accA, accB,
             ibig, rb0, rb1, rb2, rb3, ls0, ls1, ls2, ls3,
             ss0, ss1, ss2, ss3, acc_sh):
        cid = lax.axis_index("c")
        sid = lax.axis_index("s")
        rows_per = N // NS
        nch = (EPAD // NS) // GCH
        cbase = sid * nch
        rbufs = (rb0, rb1, rb2, rb3)
        lsems = (ls0, ls1, ls2, ls3)
        ssems = (ss0, ss1, ss2, ss3)

        def do_half(w_h, init_h, out_h):
            pltpu.sync_copy(init_h.at[pl.ds(sid * rows_per, rows_per)],
                            acc_sh.at[pl.ds(sid * rows_per, rows_per)])
            pltpu.sync_copy(dst2_h.at[pl.ds(cbase, nch)], ibig)
            plsc.subcore_barrier()

            def load_start(ci, b):
                pltpu.async_copy(w_h.at[pl.ds((cbase + ci) * GCH, GCH)],
                                 rbufs[b], lsems[b])

            def load_wait(b):
                pltpu.make_async_copy(w_h.at[pl.ds(0, GCH)], rbufs[b],
                                      lsems[b]).wait()

            def scat_start(ci, b):
                pltpu.async_copy(rbufs[b], acc_sh.at[ibig.at[ci]], ssems[b],
                                 add=True)

            def scat_wait(b):
                pltpu.make_async_copy(w_h.at[pl.ds(0, GCH)], rbufs[b],
                                      ssems[b]).wait()

            load_start(0, 0)

            def outer(i4, _):
                for b in range(4):
                    ci = i4 * 4 + b
                    bn = (b + 1) % 4

                    @pl.when(ci >= 3)
                    def _():
                        scat_wait(bn)

                    @pl.when(ci + 1 < nch)
                    def _():
                        load_start(ci + 1, bn)
                    load_wait(b)
                    scat_start(ci, b)
                return 0
            lax.fori_loop(0, nch // 4, outer, 0)
            scat_wait(1)
            scat_wait(2)
            scat_wait(3)
            plsc.subcore_barrier()
            pltpu.sync_copy(acc_sh.at[pl.ds(sid * rows_per, rows_per)],
                            out_h.at[pl.ds(sid * rows_per, rows_per)])

        @pl.when(cid == 0)
        def _():
            do_half(wA, initA, accA)

        @pl.when(cid == 1)
        def _():
            do_half(wB, initB, accB)

    return pl.kernel(
        body,
        out_type=outs,
        mesh=_mesh(),
        scratch_types=[
            pltpu.VMEM((EPAD // GCH // NS, GCH), _i32),
            pltpu.VMEM((GCH, QC), _f32),
            pltpu.VMEM((GCH, QC), _f32),
            pltpu.VMEM((GCH, QC), _f32),
            pltpu.VMEM((GCH, QC), _f32),
            pltpu.SemaphoreType.DMA,
            pltpu.SemaphoreType.DMA,
            pltpu.SemaphoreType.DMA,
            pltpu.SemaphoreType.DMA,
            pltpu.SemaphoreType.DMA,
            pltpu.SemaphoreType.DMA,
            pltpu.SemaphoreType.DMA,
            pltpu.SemaphoreType.DMA,
            pltpu.VMEM_SHARED((N, QC), _f32),
        ],
        compiler_params=_SC_PARAMS,
        interpret=interpret,
    )


def _head_logits(z, att, h0):
    cols = []
    for k in range(3):
        h = h0 + k
        zr = z[:, k * HID:(k + 1) * HID]
        cols.append(jnp.sum(zr * att[h:h + 1, :], axis=1, keepdims=True))
    return jnp.concatenate(cols, axis=1)


def _lrelu(x):
    return jnp.where(x >= 0, x, 0.2 * x)


def _m0_kernel(x_ref, ntm_ref, ls_ref, w4_ref, wl_ref, bl_ref, wr_ref,
               br_ref, we_ref, att_ref,
               xlA_ref, xlB_ref, xrA_ref, xrB_ref, ps_ref, la_ref):
    xb = x_ref[...]
    proj = jnp.dot(xb, w4_ref[...], preferred_element_type=_f32)
    t = ntm_ref[...]
    h0 = jnp.zeros((BN, HID), _f32)
    for k in range(4):
        h0 = jnp.where(t == float(k), proj[:, k * HID:(k + 1) * HID], h0)
    h0 = jnp.maximum(h0, 0.0)
    xl = jnp.dot(h0, wl_ref[...], preferred_element_type=_f32) + bl_ref[...]
    xr = jnp.dot(h0, wr_ref[...], preferred_element_type=_f32) + br_ref[...]
    ls = ls_ref[...]
    cnt = jnp.maximum(ls[:, 4:5], 1.0)
    la = ls / cnt
    la = jnp.where(jnp.arange(8)[None, :] < 4, la, 0.0)
    eemb = jnp.dot(la, we_ref[...], preferred_element_type=_f32)
    z = _lrelu(xl + xr + eemb)
    att = att_ref[...]
    lg = jnp.concatenate(
        [_head_logits(z[:, :HALF], att, 0), _head_logits(z[:, HALF:], att, 3)],
        axis=1)
    ps = jnp.exp(lg)
    ps_ref[...] = jnp.concatenate([ps, jnp.zeros((BN, 2), _f32)], axis=1)
    la_ref[...] = la
    xlA_ref[...] = xl[:, :HALF]
    xlB_ref[...] = xl[:, HALF:]
    xrA_ref[...] = xr[:, :HALF]
    xrB_ref[...] = xr[:, HALF:]


def _m1_kernel(e_const, glA_ref, glB_ref, grA_ref, grB_ref, ea_ref, we_ref,
               att_ref, p_ref):
    i = pl.program_id(0)
    ea = ea_ref[...]
    att = att_ref[...]
    we = we_ref[...]
    zA = _lrelu(glA_ref[...] + grA_ref[...] +
                jnp.dot(ea, we[:, :HALF], preferred_element_type=_f32))
    zB = _lrelu(glB_ref[...] + grB_ref[...] +
                jnp.dot(ea, we[:, HALF:], preferred_element_type=_f32))
    lg = jnp.concatenate([_head_logits(zA, att, 0), _head_logits(zB, att, 3)],
                         axis=1)
    p = jnp.exp(lg)
    ids = i * BE + lax.broadcasted_iota(_i32, (BE, 1), 0)
    p = jnp.where(ids < e_const, p, 0.0)
    p_ref[...] = jnp.concatenate([p, jnp.zeros((BE, 2), _f32)], axis=1)


def _m2a_kernel(ps_ref, den_ref, xlA_ref, xlB_ref,
                iA0_ref, iA1_ref, iB0_ref, iB1_ref):
    a = ps_ref[...] / jnp.maximum(den_ref[...], 1e-30)
    xlA = xlA_ref[...]
    xlB = xlB_ref[...]
    iA = jnp.concatenate(
        [a[:, h:h + 1] * xlA[:, h * HID:(h + 1) * HID] for h in range(3)], axis=1)
    iB = jnp.concatenate(
        [a[:, 3 + h:4 + h] * xlB[:, h * HID:(h + 1) * HID] for h in range(3)],
        axis=1)
    iA0_ref[...] = iA[:, :QC]
    iA1_ref[...] = iA[:, QC:]
    iB0_ref[...] = iB[:, :QC]
    iB1_ref[...] = iB[:, QC:]


def _m2c_kernel(p_ref, denE_ref, glA_ref, glB_ref,
                wA0_ref, wA1_ref, wB0_ref, wB1_ref):
    a = p_ref[...] / jnp.maximum(denE_ref[...], 1e-30)
    glA = glA_ref[...]
    glB = glB_ref[...]
    wA = jnp.concatenate(
        [a[:, h:h + 1] * glA[:, h * HID:(h + 1) * HID] for h in range(3)], axis=1)
    wB = jnp.concatenate(
        [a[:, 3 + h:4 + h] * glB[:, h * HID:(h + 1) * HID] for h in range(3)],
        axis=1)
    wA0_ref[...] = wA[:, :QC]
    wA1_ref[...] = wA[:, QC:]
    wB0_ref[...] = wB[:, :QC]
    wB1_ref[...] = wB[:, QC:]


def _ml2_body(accA0_ref, accA1_ref, accB0_ref, accB1_ref, ls_ref, bias_ref,
              wl_ref, bl_ref, wr_ref, br_ref, we_ref, att_ref,
              xlA_ref, xlB_ref, xrA_ref, xrB_ref, ps_ref):
    bias = bias_ref[...]
    accA = jnp.concatenate([accA0_ref[...], accA1_ref[...]], axis=1)
    accB = jnp.concatenate([accB0_ref[...], accB1_ref[...]], axis=1)
    h1A = jnp.maximum(accA + bias[:, :HALF], 0.0)
    h1B = jnp.maximum(accB + bias[:, HALF:], 0.0)
    wl = wl_ref[...]
    wr = wr_ref[...]
    xl = (jnp.dot(h1A, wl[:HALF, :], preferred_element_type=_f32) +
          jnp.dot(h1B, wl[HALF:, :], preferred_element_type=_f32) + bl_ref[...])
    xr = (jnp.dot(h1A, wr[:HALF, :], preferred_element_type=_f32) +
          jnp.dot(h1B, wr[HALF:, :], preferred_element_type=_f32) + br_ref[...])
    la = ls_ref[...]
    eemb = jnp.dot(la, we_ref[...], preferred_element_type=_f32)
    z = _lrelu(xl + xr + eemb)
    att = att_ref[...]
    lg = jnp.concatenate(
        [_head_logits(z[:, :HALF], att, 0), _head_logits(z[:, HALF:], att, 3)],
        axis=1)
    ps_ref[...] = jnp.concatenate([jnp.exp(lg), jnp.zeros((BN, 2), _f32)],
                                  axis=1)
    xlA_ref[...] = xl[:, :HALF]
    xlB_ref[...] = xl[:, HALF:]
    xrA_ref[...] = xr[:, :HALF]
    xrB_ref[...] = xr[:, HALF:]


def _m3_kernel(accA0_ref, accA1_ref, accB0_ref, accB1_ref, bias_ref,
               batch_ref, wo_ref, bo_ref, out_ref, pA_ref, pB_ref, cnt_ref):
    i = pl.program_id(0)
    nb = pl.num_programs(0)

    @pl.when(i == 0)
    def _():
        pA_ref[...] = jnp.zeros_like(pA_ref)
        pB_ref[...] = jnp.zeros_like(pB_ref)
        cnt_ref[...] = jnp.zeros_like(cnt_ref)

    bias = bias_ref[...]
    accA = jnp.concatenate([accA0_ref[...], accA1_ref[...]], axis=1)
    accB = jnp.concatenate([accB0_ref[...], accB1_ref[...]], axis=1)
    h2A = jnp.maximum(accA + bias[:, :HALF], 0.0)
    h2B = jnp.maximum(accB + bias[:, HALF:], 0.0)
    b = batch_ref[...]
    gids = lax.broadcasted_iota(_i32, (1, NG), 1).astype(_f32)
    oh = (b == gids).astype(_f32)
    dn = (((0,), (0,)), ((), ()))
    pA_ref[...] += lax.dot_general(oh, h2A, dn, preferred_element_type=_f32)
    pB_ref[...] += lax.dot_general(oh, h2B, dn, preferred_element_type=_f32)
    cnt_ref[...] += jnp.sum(oh, axis=0)[:, None]

    @pl.when(i == nb - 1)
    def _():
        cnt = jnp.maximum(cnt_ref[...], 1.0)
        wo = wo_ref[...]
        out = (jnp.dot(pA_ref[...] / cnt, wo[:HALF, :],
                       preferred_element_type=_f32) +
               jnp.dot(pB_ref[...] / cnt, wo[HALF:, :],
                       preferred_element_type=_f32) + bo_ref[...])
        out_ref[...] = jnp.tanh(out)


_NB = N // BN
_EB = EPAD // BE


def _nspec(c):
    return pl.BlockSpec((BN, c), lambda i: (i, 0))


def _espec(c):
    return pl.BlockSpec((BE, c), lambda i: (i, 0))


def _full(shape):
    return pl.BlockSpec(shape, lambda i: tuple(0 for _ in shape))


def _m0_call(x, ntmf, ls, w4, wl, bl, wr, br, we, att, interpret=False):
    outs = (
        jax.ShapeDtypeStruct((N, HALF), _f32),
        jax.ShapeDtypeStruct((N, HALF), _f32),
        jax.ShapeDtypeStruct((N, HALF), _f32),
        jax.ShapeDtypeStruct((N, HALF), _f32),
        jax.ShapeDtypeStruct((N, 8), _f32),
        jax.ShapeDtypeStruct((N, 8), _f32),
    )
    return pl.pallas_call(
        _m0_kernel,
        grid=(_NB,),
        in_specs=[_nspec(IN_DIM), _nspec(1), _nspec(8), _full((IN_DIM, 4 * HID)),
                  _full((HID, HC)), _full((1, HC)), _full((HID, HC)),
                  _full((1, HC)), _full((8, HC)), _full((8, HID))],
        out_specs=(_nspec(HALF), _nspec(HALF), _nspec(HALF), _nspec(HALF),
                   _nspec(8), _nspec(8)),
        out_shape=outs,
        interpret=interpret,
    )(x, ntmf, ls, w4, wl, bl, wr, br, we, att)


def _m1_call(glA, glB, grA, grB, ea8, we, att, interpret=False):
    return pl.pallas_call(
        functools.partial(_m1_kernel, E),
        grid=(_EB,),
        in_specs=[_espec(HALF)] * 4 + [_espec(8), _full((8, HC)),
                                       _full((8, HID))],
        out_specs=_espec(8),
        out_shape=jax.ShapeDtypeStruct((EPAD, 8), _f32),
        interpret=interpret,
    )(glA, glB, grA, grB, ea8, we, att)


def _m2a_call(ps, den, xlA, xlB, interpret=False):
    outs = tuple(jax.ShapeDtypeStruct((N, QC), _f32) for _ in range(4))
    return pl.pallas_call(
        _m2a_kernel,
        grid=(_NB,),
        in_specs=[_nspec(8), _nspec(8), _nspec(HALF), _nspec(HALF)],
        out_specs=tuple(_nspec(QC) for _ in range(4)),
        out_shape=outs,
        interpret=interpret,
    )(ps, den, xlA, xlB)


def _m2c_call(p, denE, glA, glB, interpret=False):
    outs = tuple(jax.ShapeDtypeStruct((EPAD, QC), _f32) for _ in range(4))
    return pl.pallas_call(
        _m2c_kernel,
        grid=(_EB,),
        in_specs=[_espec(8), _espec(8), _espec(HALF), _espec(HALF)],
        out_specs=tuple(_espec(QC) for _ in range(4)),
        out_shape=outs,
        interpret=interpret,
    )(p, denE, glA, glB)


def _ml2_call(accs, ls, bias, wl, bl, wr, br, we, att, interpret=False):
    outs = tuple(jax.ShapeDtypeStruct((N, HALF), _f32) for _ in range(4)) + (
        jax.ShapeDtypeStruct((N, 8), _f32),)
    return pl.pallas_call(
        _ml2_body,
        grid=(_NB,),
        in_specs=[_nspec(QC)] * 4 + [_nspec(8), _full((1, HC)),
                  _full((HC, HC)), _full((1, HC)), _full((HC, HC)),
                  _full((1, HC)), _full((8, HC)), _full((8, HID))],
        out_specs=(_nspec(HALF), _nspec(HALF), _nspec(HALF), _nspec(HALF),
                   _nspec(8)),
        out_shape=outs,
        interpret=interpret,
    )(*accs, ls, bias, wl, bl, wr, br, we, att)


def _m3_call(accs, bias, batchf, wo, bo, interpret=False):
    return pl.pallas_call(
        _m3_kernel,
        grid=(_NB,),
        in_specs=[_nspec(QC)] * 4 + [_full((1, HC)), _nspec(1),
                  _full((HC, OUT_DIM)), _full((1, OUT_DIM))],
        out_specs=_full((NG, OUT_DIM)),
        out_shape=jax.ShapeDtypeStruct((NG, OUT_DIM), _f32),
        scratch_shapes=[pltpu.VMEM((NG, HALF), _f32),
                        pltpu.VMEM((NG, HALF), _f32),
                        pltpu.VMEM((NG, 1), _f32)],
        interpret=interpret,
    )(*accs, bias, batchf, wo, bo)


def _layer(h_parts, src_p, dst_p, dst2, ridx, ea8, la, p, first, x=None,
           ntmf=None, ls=None, w4=None, interpret=False):
    wl, bl = p["Wl"], p["bl"][None, :]
    wr, br = p["Wr"], p["br"][None, :]
    we8 = jnp.pad(p["We"], ((0, 4), (0, 0)))
    att8 = jnp.pad(p["att"], ((0, 2), (0, 0)))
    if first:
        xlA, xlB, xrA, xrB, ps, la_out = _m0_call(
            x, ntmf, ls, w4, wl, bl, wr, br, we8, att8, interpret=interpret)
    else:
        accs, bias_prev = h_parts
        xlA, xlB, xrA, xrB, ps = _ml2_call(
            accs, la, bias_prev, wl, bl, wr, br, we8, att8,
            interpret=interpret)
        la_out = la
    glA, glB, grA, grB = _gather2_build(interpret=interpret)(
        xlA, xlB, xrA, xrB, src_p, dst_p)
    pmat = _m1_call(glA, glB, grA, grB, ea8, we8, att8, interpret=interpret)
    den, denE_flat = _seg8_build(True, True, interpret=interpret)(
        pmat.reshape(-1), dst_p, jnp.zeros((N, 8), _f32), ridx, ps)
    denE = denE_flat.reshape(EPAD, 8)
    iA0, iA1, iB0, iB1 = _m2a_call(ps, den, xlA, xlB, interpret=interpret)
    wA0, wA1, wB0, wB1 = _m2c_call(pmat, denE, glA, glB, interpret=interpret)
    scat = _scatter_rows_build(interpret=interpret)
    accA0, accB0 = scat(wA0, wB0, dst2, iA0, iB0)
    accA1, accB1 = scat(wA1, wB1, dst2, iA1, iB1)
    return (accA0, accA1, accB0, accB1), la_out


def kernel(x, edge_index, edge_attr, node_type_mask, batch, params):
    src = edge_index[0].astype(_i32)
    dst = edge_index[1].astype(_i32)
    src_p = jnp.pad(src, (0, EPAD - E))
    dst_p = jnp.pad(dst, (0, EPAD - E))
    ea8 = jnp.pad(edge_attr.astype(_f32), ((0, EPAD - E), (0, 4)))
    vals0 = jnp.pad(
        jnp.concatenate([edge_attr.astype(_f32),
                         jnp.ones((E, 1), _f32)], axis=1),
        ((0, EPAD - E), (0, 3)))
    ntmf = node_type_mask.astype(_f32)[:, None]
    batchf = batch.astype(_f32)[:, None]
    w4 = jnp.concatenate(
        [params["W_" + n] for n in ["joint", "obj", "tcp", "goal"]], axis=1)

    dst2 = dst_p.reshape(EPAD // GCH, GCH)
    ridx = jnp.arange(N, dtype=_i32).reshape(_NRC, _RCH)

    interpret = False
    ls = _seg8_build(False, False, interpret=interpret)(
        vals0.reshape(-1), dst_p, jnp.zeros((N, 8), _f32), ridx)

    c0, c1 = params["convs"]
    accs, la = _layer(None, src_p, dst_p, dst2, ridx, ea8, None, c0, True,
                      x=x, ntmf=ntmf, ls=ls, w4=w4, interpret=interpret)
    accs, _ = _layer((accs, c0["bias"][None, :]), src_p, dst_p, dst2, ridx,
                     ea8, la, c1, False, interpret=interpret)
    return _m3_call(accs, c1["bias"][None, :], batchf,
                    params["W_out"], params["b_out"][None, :],
                    interpret=interpret)
